# Initial kernel scaffold; baseline (speedup 1.0000x reference)
import jax
import jax.numpy as jnp
from jax.experimental import pallas as pl


def kernel(x, edge_index, edge_attr, W1, b1, W2, b2, Wd, bd, Wu1, bu1, Wu2, bu2):
    raise NotImplementedError("write your pallas kernel here")


# speedup vs baseline: 2.8721x; 2.8721x over previous
import functools

import jax
import jax.numpy as jnp
from jax import lax
from jax.experimental import pallas as pl
from jax.experimental.pallas import tpu as pltpu
from jax.experimental.pallas import tpu_sc as plsc

N_NODES = 10000
N_EDGES = 320000
HIDDEN = 128
N_RBF = 16

LANES = 16
CHUNK = 128
EDGE_BLOCK = 6400


def _filters_body(ea_ref, w1t_ref, b1_ref, w2t_ref, b2_ref, out_ref):
    ea = ea_ref[...]
    h = jnp.dot(ea, w1t_ref[...], preferred_element_type=jnp.float32)
    h = h + b1_ref[...][None, :]
    h = h * jax.nn.sigmoid(h)
    f = jnp.dot(h, w2t_ref[...], preferred_element_type=jnp.float32)
    out_ref[...] = f + b2_ref[...][None, :]


def _filters(edge_attr, W1t, b1, W2t, b2):
    grid = N_EDGES // EDGE_BLOCK
    return pl.pallas_call(
        _filters_body,
        grid=(grid,),
        in_specs=[
            pl.BlockSpec((EDGE_BLOCK, N_RBF), lambda i: (i, 0)),
            pl.BlockSpec((N_RBF, HIDDEN), lambda i: (0, 0)),
            pl.BlockSpec((HIDDEN,), lambda i: (0,)),
            pl.BlockSpec((HIDDEN, HIDDEN), lambda i: (0, 0)),
            pl.BlockSpec((HIDDEN,), lambda i: (0,)),
        ],
        out_specs=pl.BlockSpec((EDGE_BLOCK, HIDDEN), lambda i: (i, 0)),
        out_shape=jax.ShapeDtypeStruct((N_EDGES, HIDDEN), jnp.float32),
    )(edge_attr, W1t, b1, W2t, b2)


def _xd_body(x_ref, wdt_ref, bd_ref, out_ref):
    out_ref[...] = (
        jnp.dot(x_ref[...], wdt_ref[...], preferred_element_type=jnp.float32)
        + bd_ref[...][None, :]
    )


def _xd(x, Wdt, bd):
    return pl.pallas_call(
        _xd_body,
        out_shape=jax.ShapeDtypeStruct((N_NODES, HIDDEN), jnp.float32),
    )(x, Wdt, bd)


def _update_body(x_ref, p_ref, wu1t_ref, bu1_ref, wu2t_ref, bu2_ref, out_ref):
    agg = p_ref[0, :N_NODES, :] + p_ref[1, :N_NODES, :]
    u = jnp.dot(agg, wu1t_ref[...], preferred_element_type=jnp.float32)
    u = u + bu1_ref[...][None, :]
    u = u * jax.nn.sigmoid(u)
    u = jnp.dot(u, wu2t_ref[...], preferred_element_type=jnp.float32)
    out_ref[...] = x_ref[...] + u + bu2_ref[...][None, :]


def _update(x, partials, Wu1t, bu1, Wu2t, bu2):
    return pl.pallas_call(
        _update_body,
        out_shape=jax.ShapeDtypeStruct((N_NODES, HIDDEN), jnp.float32),
    )(x, partials, Wu1t, bu1, Wu2t, bu2)


def _make_sc_aggregate():
    info = plsc.get_sparse_core_info()
    nc, ns = info.num_cores, info.num_subcores
    nw = nc * ns
    n_chunks = N_EDGES // CHUNK
    chunks_per_w = -(-n_chunks // nw)
    n_pad = ns * 640
    rows_per_tile = n_pad // ns

    mesh = plsc.VectorSubcoreMesh(core_axis_name="c", subcore_axis_name="s")

    @functools.partial(
        pl.kernel,
        mesh=mesh,
        out_type=jax.ShapeDtypeStruct((nc, n_pad, HIDDEN), jnp.float32),
        scratch_types=[
            pltpu.VMEM((CHUNK,), jnp.int32),
            pltpu.VMEM((CHUNK,), jnp.int32),
            pltpu.VMEM((CHUNK, HIDDEN), jnp.float32),
            pltpu.VMEM((CHUNK, HIDDEN), jnp.float32),
            pltpu.VMEM_SHARED((n_pad, HIDDEN), jnp.float32),
            pltpu.SemaphoreType.DMA,
        ],
    )
    def sc_agg(xd_hbm, filt_hbm, src_hbm, dst_hbm, out_hbm,
               src_v, dst_v, rows_v, filt_v, agg_sh, sem):
        cid = lax.axis_index("c")
        sid = lax.axis_index("s")
        wid = sid * nc + cid

        zeros16 = jnp.zeros((LANES,), jnp.float32)

        def zrow(r, carry):
            for j in range(HIDDEN // LANES):
                rows_v[r, pl.ds(j * LANES, LANES)] = zeros16
            return carry
        lax.fori_loop(0, CHUNK, zrow, 0)
        for i in range(rows_per_tile // CHUNK):
            pltpu.sync_copy(
                rows_v,
                agg_sh.at[pl.ds(sid * rows_per_tile + i * CHUNK, CHUNK)],
            )
        plsc.subcore_barrier()

        def chunk_body(k, carry):
            c = k * nw + wid

            @pl.when(c < n_chunks)
            def _():
                base = c * CHUNK
                pltpu.sync_copy(src_hbm.at[pl.ds(base, CHUNK)], src_v)
                pltpu.sync_copy(dst_hbm.at[pl.ds(base, CHUNK)], dst_v)
                pltpu.async_copy(xd_hbm.at[src_v], rows_v, sem).wait()
                pltpu.sync_copy(filt_hbm.at[pl.ds(base, CHUNK)], filt_v)

                def mul_row(r, cc):
                    for j in range(HIDDEN // LANES):
                        sl = pl.ds(j * LANES, LANES)
                        rows_v[r, sl] = rows_v[r, sl] * filt_v[r, sl]
                    return cc
                lax.fori_loop(0, CHUNK, mul_row, 0)

                pltpu.sync_copy(rows_v, agg_sh.at[dst_v], add=True)

            return carry

        lax.fori_loop(0, chunks_per_w, chunk_body, 0)
        plsc.subcore_barrier()

        for i in range(rows_per_tile // CHUNK):
            start = sid * rows_per_tile + i * CHUNK
            pltpu.sync_copy(
                agg_sh.at[pl.ds(start, CHUNK)],
                out_hbm.at[cid, pl.ds(start, CHUNK)],
            )

    return sc_agg


_sc_aggregate = _make_sc_aggregate()


def kernel(x, edge_index, edge_attr, W1, b1, W2, b2, Wd, bd, Wu1, bu1, Wu2, bu2):
    src = edge_index[0].astype(jnp.int32)
    dst = edge_index[1].astype(jnp.int32)
    filters = _filters(edge_attr, W1.T, b1, W2.T, b2)
    xd = _xd(x, Wd.T, bd)
    partials = _sc_aggregate(xd, filters, src, dst)
    return _update(x, partials, Wu1.T, bu1, Wu2.T, bu2)


# speedup vs baseline: 3.1218x; 1.0869x over previous
import functools

import jax
import jax.numpy as jnp
from jax import lax
from jax.experimental import pallas as pl
from jax.experimental.pallas import tpu as pltpu
from jax.experimental.pallas import tpu_sc as plsc

N_NODES = 10000
N_EDGES = 320000
HIDDEN = 128
N_RBF = 16

LANES = 16
CHUNK = 80
E_PAD = 322560
EDGE_BLOCK = 6400


def _filters_body(ea_ref, w1t_ref, b1_ref, w2t_ref, b2_ref, out_ref):
    ea = ea_ref[...]
    h = jnp.dot(ea, w1t_ref[...], preferred_element_type=jnp.float32)
    h = h + b1_ref[...][None, :]
    h = h * jax.nn.sigmoid(h)
    f = jnp.dot(h, w2t_ref[...], preferred_element_type=jnp.float32)
    out_ref[...] = f + b2_ref[...][None, :]


def _filters(edge_attr, W1t, b1, W2t, b2):
    grid = N_EDGES // EDGE_BLOCK
    return pl.pallas_call(
        _filters_body,
        grid=(grid,),
        in_specs=[
            pl.BlockSpec((EDGE_BLOCK, N_RBF), lambda i: (i, 0)),
            pl.BlockSpec((N_RBF, HIDDEN), lambda i: (0, 0)),
            pl.BlockSpec((HIDDEN,), lambda i: (0,)),
            pl.BlockSpec((HIDDEN, HIDDEN), lambda i: (0, 0)),
            pl.BlockSpec((HIDDEN,), lambda i: (0,)),
        ],
        out_specs=pl.BlockSpec((EDGE_BLOCK, HIDDEN), lambda i: (i, 0)),
        out_shape=jax.ShapeDtypeStruct((E_PAD, HIDDEN), jnp.float32),
    )(edge_attr, W1t, b1, W2t, b2)


def _xd_body(x_ref, wdt_ref, bd_ref, out_ref):
    out_ref[...] = (
        jnp.dot(x_ref[...], wdt_ref[...], preferred_element_type=jnp.float32)
        + bd_ref[...][None, :]
    )


def _xd(x, Wdt, bd):
    return pl.pallas_call(
        _xd_body,
        out_shape=jax.ShapeDtypeStruct((N_NODES, HIDDEN), jnp.float32),
    )(x, Wdt, bd)


def _update_body(x_ref, p_ref, wu1t_ref, bu1_ref, wu2t_ref, bu2_ref, out_ref):
    agg = p_ref[0, :N_NODES, :] + p_ref[1, :N_NODES, :]
    u = jnp.dot(agg, wu1t_ref[...], preferred_element_type=jnp.float32)
    u = u + bu1_ref[...][None, :]
    u = u * jax.nn.sigmoid(u)
    u = jnp.dot(u, wu2t_ref[...], preferred_element_type=jnp.float32)
    out_ref[...] = x_ref[...] + u + bu2_ref[...][None, :]


def _update(x, partials, Wu1t, bu1, Wu2t, bu2):
    return pl.pallas_call(
        _update_body,
        out_shape=jax.ShapeDtypeStruct((N_NODES, HIDDEN), jnp.float32),
    )(x, partials, Wu1t, bu1, Wu2t, bu2)


def _make_sc_aggregate():
    info = plsc.get_sparse_core_info()
    nc, ns = info.num_cores, info.num_subcores
    nw = nc * ns
    n_chunks = E_PAD // CHUNK
    chunks_per_w = n_chunks // nw
    n_pad = ns * 640
    rows_per_tile = n_pad // ns
    assert rows_per_tile % CHUNK == 0 and chunks_per_w % 2 == 0

    mesh = plsc.VectorSubcoreMesh(core_axis_name="c", subcore_axis_name="s")

    @functools.partial(
        pl.kernel,
        mesh=mesh,
        out_type=jax.ShapeDtypeStruct((nc, n_pad, HIDDEN), jnp.float32),
        scratch_types=[
            pltpu.VMEM((2, CHUNK), jnp.int32),
            pltpu.VMEM((2, CHUNK), jnp.int32),
            pltpu.VMEM((2, CHUNK, HIDDEN), jnp.float32),
            pltpu.VMEM((2, CHUNK, HIDDEN), jnp.float32),
            pltpu.VMEM_SHARED((n_pad, HIDDEN), jnp.float32),
            pltpu.SemaphoreType.DMA,
            pltpu.SemaphoreType.DMA,
            pltpu.SemaphoreType.DMA,
            pltpu.SemaphoreType.DMA,
        ],
    )
    def sc_agg(xd_hbm, filt_hbm, src_hbm, dst_hbm, out_hbm,
               src_v, dst_v, rows_v, filt_v, agg_sh,
               sem_l0, sem_l1, sem_g0, sem_g1):
        cid = lax.axis_index("c")
        sid = lax.axis_index("s")
        wid = sid * nc + cid
        sem_l = (sem_l0, sem_l1)
        sem_g = (sem_g0, sem_g1)

        zeros16 = jnp.zeros((LANES,), jnp.float32)

        def zrow(r, carry):
            for j in range(HIDDEN // LANES):
                rows_v[0, r, pl.ds(j * LANES, LANES)] = zeros16
            return carry
        lax.fori_loop(0, CHUNK, zrow, 0)
        for i in range(rows_per_tile // CHUNK):
            pltpu.sync_copy(
                rows_v.at[0],
                agg_sh.at[pl.ds(sid * rows_per_tile + i * CHUNK, CHUNK)],
            )
        plsc.subcore_barrier()

        def chunk_base(k):
            return (k * nw + wid) * CHUNK

        def start_loads(k, b):
            base = chunk_base(k)
            pltpu.async_copy(src_hbm.at[pl.ds(base, CHUNK)], src_v.at[b], sem_l[b])
            pltpu.async_copy(dst_hbm.at[pl.ds(base, CHUNK)], dst_v.at[b], sem_l[b])
            pltpu.async_copy(filt_hbm.at[pl.ds(base, CHUNK)], filt_v.at[b], sem_l[b])

        def wait_loads(k, b):
            base = chunk_base(k)
            pltpu.make_async_copy(src_hbm.at[pl.ds(base, CHUNK)], src_v.at[b], sem_l[b]).wait()
            pltpu.make_async_copy(dst_hbm.at[pl.ds(base, CHUNK)], dst_v.at[b], sem_l[b]).wait()
            pltpu.make_async_copy(filt_hbm.at[pl.ds(base, CHUNK)], filt_v.at[b], sem_l[b]).wait()

        def start_gather(b):
            pltpu.async_copy(xd_hbm.at[src_v.at[b]], rows_v.at[b], sem_g[b])

        def wait_gather(b):
            pltpu.make_async_copy(xd_hbm.at[src_v.at[b]], rows_v.at[b], sem_g[b]).wait()

        start_loads(0, 0)
        start_loads(1, 1)
        wait_loads(0, 0)
        start_gather(0)

        def body2(i2, carry):
            for b in (0, 1):
                k = i2 * 2 + b
                nb = 1 - b

                @pl.when(k + 1 < chunks_per_w)
                def _():
                    wait_loads(k + 1, nb)
                    start_gather(nb)

                wait_gather(b)

                def mul4(r4, cc):
                    r = r4 * 4
                    for dr in range(4):
                        for j in range(HIDDEN // LANES):
                            sl = pl.ds(j * LANES, LANES)
                            rows_v[b, r + dr, sl] = (
                                rows_v[b, r + dr, sl] * filt_v[b, r + dr, sl]
                            )
                    return cc
                lax.fori_loop(0, CHUNK // 4, mul4, 0)

                pltpu.sync_copy(rows_v.at[b], agg_sh.at[dst_v.at[b]], add=True)

                @pl.when(k + 2 < chunks_per_w)
                def _():
                    start_loads(k + 2, b)
            return carry

        lax.fori_loop(0, chunks_per_w // 2, body2, 0)
        plsc.subcore_barrier()

        for i in range(rows_per_tile // CHUNK):
            start = sid * rows_per_tile + i * CHUNK
            pltpu.sync_copy(
                agg_sh.at[pl.ds(start, CHUNK)],
                out_hbm.at[cid, pl.ds(start, CHUNK)],
            )

    return sc_agg


_sc_aggregate = _make_sc_aggregate()


def kernel(x, edge_index, edge_attr, W1, b1, W2, b2, Wd, bd, Wu1, bu1, Wu2, bu2):
    src = edge_index[0].astype(jnp.int32)
    dst = edge_index[1].astype(jnp.int32)
    pad = E_PAD - N_EDGES
    src = jnp.concatenate([src, jnp.zeros((pad,), jnp.int32)])
    dst = jnp.concatenate([dst, jnp.full((pad,), N_NODES, jnp.int32)])
    filters = _filters(edge_attr, W1.T, b1, W2.T, b2)
    xd = _xd(x, Wd.T, bd)
    partials = _sc_aggregate(xd, filters, src, dst)
    return _update(x, partials, Wu1.T, bu1, Wu2.T, bu2)
